# baseline (speedup 1.0000x reference)
import functools

import jax
import jax.numpy as jnp
from jax import lax
from jax.experimental import pallas as pl
from jax.experimental.pallas import tpu as pltpu
from jax.experimental.pallas import tpu_sc as plsc

_VOCAB = 100000
_D = 128
_BATCH = 4096
_HIST = 50
_TOT = _BATCH * _HIST

_NC = 2
_NS = 16
_NW = _NC * _NS
_CB = _BATCH // _NW

_NCH = _HIST
_NHALF = _NCH // 2


def _gather_body(idx_hbm, table_hbm, out_hbm,
                 idx0_v, idx1_v, rows0_v, rows1_v,
                 gsem0, gsem1, osem0, osem1):
    wid = lax.axis_index("s") * _NC + lax.axis_index("c")
    col0 = wid * _CB

    def fire_gather(h, idx_v, rows_v, sem):
        pltpu.sync_copy(idx_hbm.at[h, pl.ds(col0, _CB)], idx_v)
        return pltpu.async_copy(table_hbm.at[idx_v], rows_v, sem)

    def drain_gather(rows_v, sem):
        pltpu.make_async_copy(table_hbm.at[pl.ds(0, _CB)], rows_v, sem).wait()

    def out_slice(h):
        return out_hbm.at[pl.ds(h * _BATCH + col0, _CB)]

    def pair(i, prefetch):
        h0 = 2 * i
        cp1 = fire_gather(h0 + 1, idx1_v, rows1_v, gsem1)
        drain_gather(rows0_v, gsem0)
        ocp0 = pltpu.async_copy(rows0_v, out_slice(h0), osem0)
        cp1.wait()
        ocp1 = pltpu.async_copy(rows1_v, out_slice(h0 + 1), osem1)
        ocp0.wait()
        if prefetch:
            fire_gather(h0 + 2, idx0_v, rows0_v, gsem0)
        ocp1.wait()

    fire_gather(0, idx0_v, rows0_v, gsem0)

    @pl.loop(0, _NHALF - 1)
    def _pair(i):
        pair(i, prefetch=True)

    pair(_NHALF - 1, prefetch=False)


_sc_gather = functools.partial(
    pl.kernel,
    out_type=jax.ShapeDtypeStruct((_TOT, _D), jnp.float32),
    mesh=plsc.VectorSubcoreMesh(core_axis_name="c", subcore_axis_name="s"),
    scratch_types=[
        pltpu.VMEM((_CB,), jnp.int32),
        pltpu.VMEM((_CB,), jnp.int32),
        pltpu.VMEM((_CB, _D), jnp.float32),
        pltpu.VMEM((_CB, _D), jnp.float32),
        pltpu.SemaphoreType.DMA,
        pltpu.SemaphoreType.DMA,
        pltpu.SemaphoreType.DMA,
        pltpu.SemaphoreType.DMA,
    ],
    compiler_params=pltpu.CompilerParams(use_tc_tiling_on_sc=True),
)(_gather_body)


@jax.jit
def kernel(x, embeddings):
    idx = x.astype(jnp.int32).T
    out = _sc_gather(idx, embeddings)
    return out.reshape(_HIST, _BATCH, _D).transpose(1, 0, 2)


# speedup vs baseline: 1.0725x; 1.0725x over previous
import functools

import jax
import jax.numpy as jnp
from jax import lax
from jax.experimental import pallas as pl
from jax.experimental.pallas import tpu as pltpu
from jax.experimental.pallas import tpu_sc as plsc

_VOCAB = 100000
_D = 128
_BATCH = 4096
_HIST = 50
_TOT = _BATCH * _HIST

_NC = 2
_NS = 16
_NW = _NC * _NS

_C = 256
_CPH = _BATCH // _C
_NCH = _TOT // (_C * _NW)
_NHALF = (_NCH - 1) // 2
_SPLITS = [(0, 128), (128, 128)]


def _gather_body(idx_hbm, table_hbm, out_hbm,
                 idx0_v, idx1_v, rows0_v, rows1_v,
                 gsem0, gsem1, osem0, osem1):
    wid = lax.axis_index("s") * _NC + lax.axis_index("c")
    c0 = wid * _NCH

    def fire_gather(g, idx_v, rows_v, sem):
        c = c0 + g
        h = c // _CPH
        col = (c % _CPH) * _C
        pltpu.sync_copy(idx_hbm.at[h, pl.ds(col, _C)], idx_v)
        return [
            pltpu.async_copy(
                table_hbm.at[idx_v.at[pl.ds(s, n)]],
                rows_v.at[pl.ds(s, n)],
                sem,
            )
            for s, n in _SPLITS
        ]

    def drain_gather(rows_v, sem):
        pltpu.make_async_copy(table_hbm.at[pl.ds(0, _C)], rows_v, sem).wait()

    def store(g, rows_v, sem):
        return pltpu.async_copy(
            rows_v, out_hbm.at[pl.ds((c0 + g) * _C, _C)], sem)

    def pair(i):
        g0 = 2 * i
        cps1 = fire_gather(g0 + 1, idx1_v, rows1_v, gsem1)
        drain_gather(rows0_v, gsem0)
        ocp0 = store(g0, rows0_v, osem0)
        for cp in cps1:
            cp.wait()
        ocp1 = store(g0 + 1, rows1_v, osem1)
        ocp0.wait()
        fire_gather(g0 + 2, idx0_v, rows0_v, gsem0)
        ocp1.wait()

    fire_gather(0, idx0_v, rows0_v, gsem0)

    @pl.loop(0, _NHALF)
    def _pair(i):
        pair(i)

    drain_gather(rows0_v, gsem0)
    store(_NCH - 1, rows0_v, osem0).wait()


_sc_gather = functools.partial(
    pl.kernel,
    out_type=jax.ShapeDtypeStruct((_TOT, _D), jnp.float32),
    mesh=plsc.VectorSubcoreMesh(core_axis_name="c", subcore_axis_name="s"),
    scratch_types=[
        pltpu.VMEM((_C,), jnp.int32),
        pltpu.VMEM((_C,), jnp.int32),
        pltpu.VMEM((_C, _D), jnp.float32),
        pltpu.VMEM((_C, _D), jnp.float32),
        pltpu.SemaphoreType.DMA,
        pltpu.SemaphoreType.DMA,
        pltpu.SemaphoreType.DMA,
        pltpu.SemaphoreType.DMA,
    ],
    compiler_params=pltpu.CompilerParams(use_tc_tiling_on_sc=True),
)(_gather_body)


@jax.jit
def kernel(x, embeddings):
    idx = x.astype(jnp.int32).T
    out = _sc_gather(idx, embeddings)
    return out.reshape(_HIST, _BATCH, _D).transpose(1, 0, 2)


# speedup vs baseline: 1.0746x; 1.0019x over previous
import functools

import jax
import jax.numpy as jnp
from jax import lax
from jax.experimental import pallas as pl
from jax.experimental.pallas import tpu as pltpu
from jax.experimental.pallas import tpu_sc as plsc

_VOCAB = 100000
_D = 128
_BATCH = 4096
_HIST = 50
_TOT = _BATCH * _HIST

_NC = 2
_NS = 16
_NW = _NC * _NS

_C = 256
_CPH = _BATCH // _C
_NCH = _TOT // (_C * _NW)
_NHALF = (_NCH - 1) // 2
_SPLITS = [(0, 128), (128, 128)]


def _gather_body(idx_hbm, table_hbm, out_hbm,
                 idx0_v, idx1_v, rows0_v, rows1_v,
                 gsem0, gsem1, osem0, osem1):
    wid = lax.axis_index("s") * _NC + lax.axis_index("c")
    c0 = wid * _NCH

    def fire_gather(g, idx_v, rows_v, sem):
        c = c0 + g
        h = c // _CPH
        col = (c % _CPH) * _C
        pltpu.sync_copy(idx_hbm.at[h, pl.ds(col, _C)], idx_v)
        return [
            pltpu.async_copy(
                table_hbm.at[idx_v.at[pl.ds(s, n)]],
                rows_v.at[pl.ds(s, n)],
                sem,
            )
            for s, n in _SPLITS
        ]

    def drain_gather(rows_v, sem):
        pltpu.make_async_copy(table_hbm.at[pl.ds(0, _C)], rows_v, sem).wait()

    def store(g, rows_v, sem):
        return pltpu.async_copy(
            rows_v, out_hbm.at[pl.ds((c0 + g) * _C, _C)], sem)

    def pair(i):
        g0 = 2 * i
        cps1 = fire_gather(g0 + 1, idx1_v, rows1_v, gsem1)
        drain_gather(rows0_v, gsem0)
        ocp0 = store(g0, rows0_v, osem0)
        for cp in cps1:
            cp.wait()
        ocp1 = store(g0 + 1, rows1_v, osem1)
        ocp0.wait()
        fire_gather(g0 + 2, idx0_v, rows0_v, gsem0)
        ocp1.wait()

    fire_gather(0, idx0_v, rows0_v, gsem0)

    @pl.loop(0, _NHALF)
    def _pair(i):
        pair(i)

    drain_gather(rows0_v, gsem0)
    store(_NCH - 1, rows0_v, osem0).wait()


_sc_gather = functools.partial(
    pl.kernel,
    out_type=jax.ShapeDtypeStruct((_TOT, _D), jnp.float32),
    mesh=plsc.VectorSubcoreMesh(core_axis_name="c", subcore_axis_name="s"),
    scratch_types=[
        pltpu.VMEM((_C,), jnp.int32),
        pltpu.VMEM((_C,), jnp.int32),
        pltpu.VMEM((_C, _D), jnp.float32),
        pltpu.VMEM((_C, _D), jnp.float32),
        pltpu.SemaphoreType.DMA,
        pltpu.SemaphoreType.DMA,
        pltpu.SemaphoreType.DMA,
        pltpu.SemaphoreType.DMA,
    ],
    compiler_params=pltpu.CompilerParams(
        use_tc_tiling_on_sc=True, skip_device_barrier=True),
)(_gather_body)


@jax.jit
def kernel(x, embeddings):
    idx = x.astype(jnp.int32).T
    out = _sc_gather(idx, embeddings)
    return out.reshape(_HIST, _BATCH, _D).transpose(1, 0, 2)


# speedup vs baseline: 1.0988x; 1.0226x over previous
import functools

import jax
import jax.numpy as jnp
from jax import lax
from jax.experimental import pallas as pl
from jax.experimental.pallas import tpu as pltpu
from jax.experimental.pallas import tpu_sc as plsc

_VOCAB = 100000
_D = 128
_BATCH = 4096
_HIST = 50
_TOT = _BATCH * _HIST

_NC = 2
_NS = 16
_NW = _NC * _NS
_BPW = _TOT // _NW

_NBUF = 3
_C = 320
_NCH = _BPW // _C
_NITER = (_NCH - 2) // _NBUF
_SPLITS = [(0, 128), (128, 128), (256, 64)]


def _gather_body(idx_hbm, table_hbm, out_hbm,
                 idx0_v, idx1_v, idx2_v, rows0_v, rows1_v, rows2_v,
                 gsem0, gsem1, gsem2, osem0, osem1, osem2):
    wid = lax.axis_index("s") * _NC + lax.axis_index("c")
    base = wid * _BPW
    idx_v = (idx0_v, idx1_v, idx2_v)
    rows_v = (rows0_v, rows1_v, rows2_v)
    gsem = (gsem0, gsem1, gsem2)
    osem = (osem0, osem1, osem2)

    def fire_gather(g, b):
        pltpu.sync_copy(idx_hbm.at[pl.ds(base + g * _C, _C)], idx_v[b])
        for s, n in _SPLITS:
            pltpu.async_copy(
                table_hbm.at[idx_v[b].at[pl.ds(s, n)]],
                rows_v[b].at[pl.ds(s, n)],
                gsem[b],
            )

    def drain_gather(b):
        pltpu.make_async_copy(
            table_hbm.at[pl.ds(0, _C)], rows_v[b], gsem[b]).wait()

    def fire_store(g, b):
        pltpu.async_copy(
            rows_v[b], out_hbm.at[pl.ds(base + g * _C, _C)], osem[b])

    def drain_store(b):
        pltpu.make_async_copy(
            rows_v[b], out_hbm.at[pl.ds(0, _C)], osem[b]).wait()

    fire_gather(0, 0)
    fire_gather(1, 1)

    @pl.loop(0, _NITER)
    def _ring(i):
        g_base = _NBUF * i
        for s in range(_NBUF):
            g = g_base + s
            t = (s + 2) % _NBUF
            drain_gather(s)
            fire_store(g, s)
            if s == 0:
                @pl.when(i > 0)
                def _():
                    drain_store(t)
            else:
                drain_store(t)
            fire_gather(g + 2, t)

    drain_gather(0)
    fire_store(_NCH - 2, 0)
    drain_store(2)
    drain_gather(1)
    fire_store(_NCH - 1, 1)
    drain_store(0)
    drain_store(1)


_sc_gather = functools.partial(
    pl.kernel,
    out_type=jax.ShapeDtypeStruct((_TOT, _D), jnp.float32),
    mesh=plsc.VectorSubcoreMesh(core_axis_name="c", subcore_axis_name="s"),
    scratch_types=(
        [pltpu.VMEM((_C,), jnp.int32)] * _NBUF
        + [pltpu.VMEM((_C, _D), jnp.float32)] * _NBUF
        + [pltpu.SemaphoreType.DMA] * (2 * _NBUF)
    ),
)(_gather_body)


@jax.jit
def kernel(x, embeddings):
    idx = x.astype(jnp.int32).T.reshape(_TOT)
    out = _sc_gather(idx, embeddings)
    return out.reshape(_HIST, _BATCH, _D).transpose(1, 0, 2)


# speedup vs baseline: 1.1017x; 1.0026x over previous
import functools

import jax
import jax.numpy as jnp
from jax import lax
from jax.experimental import pallas as pl
from jax.experimental.pallas import tpu as pltpu
from jax.experimental.pallas import tpu_sc as plsc

_VOCAB = 100000
_D = 128
_BATCH = 4096
_HIST = 50
_TOT = _BATCH * _HIST

_NC = 2
_NS = 16
_NW = _NC * _NS
_BPW = _TOT // _NW

_NBUF = 3
_C = 320
_NCH = _BPW // _C
_NITER = (_NCH - 2) // _NBUF
_SPLITS = [(0, 128), (128, 128), (256, 64)]


def _gather_body(idx_hbm, table_hbm, out_hbm,
                 idx0_v, idx1_v, idx2_v, rows0_v, rows1_v, rows2_v,
                 gsem0, gsem1, gsem2, osem0, osem1, osem2):
    wid = lax.axis_index("s") * _NC + lax.axis_index("c")
    base = wid * _BPW
    idx_v = (idx0_v, idx1_v, idx2_v)
    rows_v = (rows0_v, rows1_v, rows2_v)
    gsem = (gsem0, gsem1, gsem2)
    osem = (osem0, osem1, osem2)

    def fire_gather(g, b):
        pltpu.sync_copy(idx_hbm.at[pl.ds(base + g * _C, _C)], idx_v[b])
        for s, n in _SPLITS:
            pltpu.async_copy(
                table_hbm.at[idx_v[b].at[pl.ds(s, n)]],
                rows_v[b].at[pl.ds(s, n)],
                gsem[b],
            )

    def drain_gather(b):
        pltpu.make_async_copy(
            table_hbm.at[pl.ds(0, _C)], rows_v[b], gsem[b]).wait()

    def fire_store(g, b):
        half = _C // 2
        for s in (0, half):
            pltpu.async_copy(
                rows_v[b].at[pl.ds(s, half)],
                out_hbm.at[pl.ds(base + g * _C + s, half)],
                osem[b],
            )

    def drain_store(b):
        pltpu.make_async_copy(
            rows_v[b], out_hbm.at[pl.ds(0, _C)], osem[b]).wait()

    fire_gather(0, 0)
    fire_gather(1, 1)

    @pl.loop(0, _NITER)
    def _ring(i):
        g_base = _NBUF * i
        for s in range(_NBUF):
            g = g_base + s
            t = (s + 2) % _NBUF
            drain_gather(s)
            fire_store(g, s)
            if s == 0:
                @pl.when(i > 0)
                def _():
                    drain_store(t)
            else:
                drain_store(t)
            fire_gather(g + 2, t)

    drain_gather(0)
    fire_store(_NCH - 2, 0)
    drain_store(2)
    drain_gather(1)
    fire_store(_NCH - 1, 1)
    drain_store(0)
    drain_store(1)


_sc_gather = functools.partial(
    pl.kernel,
    out_type=jax.ShapeDtypeStruct((_TOT, _D), jnp.float32),
    mesh=plsc.VectorSubcoreMesh(core_axis_name="c", subcore_axis_name="s"),
    scratch_types=(
        [pltpu.VMEM((_C,), jnp.int32)] * _NBUF
        + [pltpu.VMEM((_C, _D), jnp.float32)] * _NBUF
        + [pltpu.SemaphoreType.DMA] * (2 * _NBUF)
    ),
)(_gather_body)


@jax.jit
def kernel(x, embeddings):
    idx = x.astype(jnp.int32).T.reshape(_TOT)
    out = _sc_gather(idx, embeddings)
    return out.reshape(_HIST, _BATCH, _D).transpose(1, 0, 2)
